# Initial kernel scaffold; baseline (speedup 1.0000x reference)
import jax
import jax.numpy as jnp
from jax.experimental import pallas as pl


def kernel(inputs, embeddings):
    raise NotImplementedError("write your pallas kernel here")


# speedup vs baseline: 9.4717x; 9.4717x over previous
import functools

import jax
import jax.numpy as jnp
from jax import lax
from jax.experimental import pallas as pl
from jax.experimental.pallas import tpu as pltpu
from jax.experimental.pallas import tpu_sc as plsc

N_EMB = 8192
DIM = 32
N_ROWS = 16384
BLOCK = 256
GRID = N_ROWS // BLOCK

SC_CORES = 2
SC_SUBCORES = 16
SC_WORKERS = SC_CORES * SC_SUBCORES
ROWS_PER_WORKER = N_ROWS // SC_WORKERS
GATHER_CHUNK = 128
CHUNKS_PER_WORKER = ROWS_PER_WORKER // GATHER_CHUNK
DIM_PAD = 128


def _tc_body(x_ref, e_ref, et_ref, idx_ref, loss_ref):
    i = pl.program_id(0)
    x = x_ref[...]
    e = e_ref[...]
    et = et_ref[...]

    xx = jnp.sum(x * x, axis=1, keepdims=True)
    ee = jnp.sum(e * e, axis=1)
    mm = jnp.dot(x, et, preferred_element_type=jnp.float32)
    dist = (xx + ee[None, :]) - 2.0 * mm

    m = jnp.min(dist, axis=1, keepdims=True)
    ii = lax.broadcasted_iota(jnp.int32, dist.shape, 1)
    idx = jnp.min(jnp.where(dist == m, ii, jnp.int32(N_EMB)), axis=1)
    idx_ref[...] = idx

    @pl.when(i == 0)
    def _init():
        loss_ref[...] = jnp.zeros((1, 1), jnp.float32)

    loss_ref[...] += jnp.sum(m).reshape(1, 1)

    @pl.when(i == GRID - 1)
    def _finalize():
        loss_ref[...] = loss_ref[...] * (1.25 / float(N_ROWS * DIM))


_tc_call = pl.pallas_call(
    _tc_body,
    grid=(GRID,),
    in_specs=[
        pl.BlockSpec((BLOCK, DIM), lambda i: (i, 0)),
        pl.BlockSpec((N_EMB, DIM), lambda i: (0, 0)),
        pl.BlockSpec((DIM, N_EMB), lambda i: (0, 0)),
    ],
    out_specs=[
        pl.BlockSpec((BLOCK,), lambda i: (i,)),
        pl.BlockSpec((1, 1), lambda i: (0, 0)),
    ],
    out_shape=[
        jax.ShapeDtypeStruct((N_ROWS,), jnp.int32),
        jax.ShapeDtypeStruct((1, 1), jnp.float32),
    ],
)


def _sc_gather_body(table_hbm, idx_hbm, out_hbm, idx_v, rows_v, sem):
    wid = lax.axis_index("s") * SC_CORES + lax.axis_index("c")
    base = wid * ROWS_PER_WORKER

    def step(j, carry):
        off = pl.multiple_of(base + j * GATHER_CHUNK, GATHER_CHUNK)
        pltpu.sync_copy(idx_hbm.at[pl.ds(off, GATHER_CHUNK)], idx_v)
        pltpu.async_copy(table_hbm.at[idx_v], rows_v, sem).wait()
        pltpu.sync_copy(rows_v, out_hbm.at[pl.ds(off, GATHER_CHUNK)])
        return carry

    lax.fori_loop(0, CHUNKS_PER_WORKER, step, 0)


@functools.cache
def _sc_gather():
    return functools.partial(
        pl.kernel,
        out_type=jax.ShapeDtypeStruct((N_ROWS, DIM_PAD), jnp.float32),
        mesh=plsc.VectorSubcoreMesh(core_axis_name="c", subcore_axis_name="s"),
        scratch_types=[
            pltpu.VMEM((GATHER_CHUNK,), jnp.int32),
            pltpu.VMEM((GATHER_CHUNK, DIM_PAD), jnp.float32),
            pltpu.SemaphoreType.DMA,
        ],
    )(_sc_gather_body)


def kernel(inputs, embeddings):
    x = inputs.reshape(N_ROWS, DIM)
    et = embeddings.T
    idx, loss = _tc_call(x, embeddings, et)
    table = jnp.pad(embeddings, ((0, 0), (0, DIM_PAD - DIM)))
    latent = _sc_gather()(table, idx)[:, :DIM]
    return loss[0, 0], latent.reshape(inputs.shape)


# speedup vs baseline: 11.2778x; 1.1907x over previous
import functools

import jax
import jax.numpy as jnp
from jax import lax
from jax.experimental import pallas as pl
from jax.experimental.pallas import tpu as pltpu
from jax.experimental.pallas import tpu_sc as plsc

N_EMB = 8192
DIM = 32
N_ROWS = 16384
BLOCK = 512
GRID = N_ROWS // BLOCK

SC_CORES = 2
SC_SUBCORES = 16
SC_WORKERS = SC_CORES * SC_SUBCORES
ROWS_PER_WORKER = N_ROWS // SC_WORKERS
GATHER_CHUNK = 128
CHUNKS_PER_WORKER = ROWS_PER_WORKER // GATHER_CHUNK
DIM_PAD = 128


def _tc_body(x_ref, e_ref, et_ref, idx_ref, loss_ref):
    i = pl.program_id(0)
    x = x_ref[...]
    e = e_ref[...]
    et = et_ref[...]

    xx = jnp.sum(x * x, axis=1, keepdims=True)
    ee = jnp.sum(e * e, axis=1)
    mm = jnp.dot(x, et, preferred_element_type=jnp.float32)
    dist = (xx + ee[None, :]) - 2.0 * mm

    m = jnp.min(dist, axis=1, keepdims=True)
    ii = lax.broadcasted_iota(jnp.int32, dist.shape, 1)
    idx = jnp.min(jnp.where(dist == m, ii, jnp.int32(N_EMB)), axis=1)
    idx_ref[...] = idx

    @pl.when(i == 0)
    def _init():
        loss_ref[...] = jnp.zeros((1, 1), jnp.float32)

    loss_ref[...] += jnp.sum(m).reshape(1, 1)

    @pl.when(i == GRID - 1)
    def _finalize():
        loss_ref[...] = loss_ref[...] * (1.25 / float(N_ROWS * DIM))


_tc_call = pl.pallas_call(
    _tc_body,
    grid=(GRID,),
    in_specs=[
        pl.BlockSpec((BLOCK, DIM), lambda i: (i, 0)),
        pl.BlockSpec((N_EMB, DIM), lambda i: (0, 0)),
        pl.BlockSpec((DIM, N_EMB), lambda i: (0, 0)),
    ],
    out_specs=[
        pl.BlockSpec((BLOCK,), lambda i: (i,)),
        pl.BlockSpec((1, 1), lambda i: (0, 0)),
    ],
    out_shape=[
        jax.ShapeDtypeStruct((N_ROWS,), jnp.int32),
        jax.ShapeDtypeStruct((1, 1), jnp.float32),
    ],
)


def _sc_gather_body(table_hbm, idx_hbm, out_hbm, idx_v, rows_v, sem):
    wid = lax.axis_index("s") * SC_CORES + lax.axis_index("c")
    base = wid * ROWS_PER_WORKER

    def step(j, carry):
        off = pl.multiple_of(base + j * GATHER_CHUNK, GATHER_CHUNK)
        pltpu.sync_copy(idx_hbm.at[pl.ds(off, GATHER_CHUNK)], idx_v)
        pltpu.async_copy(table_hbm.at[idx_v], rows_v, sem).wait()
        pltpu.sync_copy(rows_v, out_hbm.at[pl.ds(off, GATHER_CHUNK)])
        return carry

    lax.fori_loop(0, CHUNKS_PER_WORKER, step, 0)


@functools.cache
def _sc_gather():
    return functools.partial(
        pl.kernel,
        out_type=jax.ShapeDtypeStruct((N_ROWS, DIM_PAD), jnp.float32),
        mesh=plsc.VectorSubcoreMesh(core_axis_name="c", subcore_axis_name="s"),
        scratch_types=[
            pltpu.VMEM((GATHER_CHUNK,), jnp.int32),
            pltpu.VMEM((GATHER_CHUNK, DIM_PAD), jnp.float32),
            pltpu.SemaphoreType.DMA,
        ],
    )(_sc_gather_body)


def kernel(inputs, embeddings):
    x = inputs.reshape(N_ROWS, DIM)
    et = embeddings.T
    idx, loss = _tc_call(x, embeddings, et)
    table = jnp.pad(embeddings, ((0, 0), (0, DIM_PAD - DIM)))
    latent = _sc_gather()(table, idx)[:, :DIM]
    return loss[0, 0], latent.reshape(inputs.shape)


# speedup vs baseline: 12.1875x; 1.0807x over previous
import functools

import jax
import jax.numpy as jnp
from jax import lax
from jax.experimental import pallas as pl
from jax.experimental.pallas import tpu as pltpu
from jax.experimental.pallas import tpu_sc as plsc

N_EMB = 8192
DIM = 32
N_ROWS = 16384
BLOCK = 1024
GRID = N_ROWS // BLOCK

SC_CORES = 2
SC_SUBCORES = 16
SC_WORKERS = SC_CORES * SC_SUBCORES
ROWS_PER_WORKER = N_ROWS // SC_WORKERS
GATHER_CHUNK = 128
CHUNKS_PER_WORKER = ROWS_PER_WORKER // GATHER_CHUNK
DIM_PAD = 128


def _tc_body(x_ref, e_ref, et_ref, idx_ref, loss_ref):
    i = pl.program_id(0)
    x = x_ref[...]
    e = e_ref[...]
    et = et_ref[...]

    xx = jnp.sum(x * x, axis=1, keepdims=True)
    ee = jnp.sum(e * e, axis=1)
    mm = jnp.dot(x, et, preferred_element_type=jnp.float32)
    dist = (xx + ee[None, :]) - 2.0 * mm

    m = jnp.min(dist, axis=1, keepdims=True)
    ii = lax.broadcasted_iota(jnp.int32, dist.shape, 1)
    idx = jnp.min(jnp.where(dist == m, ii, jnp.int32(N_EMB)), axis=1)
    idx_ref[...] = idx

    @pl.when(i == 0)
    def _init():
        loss_ref[...] = jnp.zeros((1, 1), jnp.float32)

    loss_ref[...] += jnp.sum(m).reshape(1, 1)

    @pl.when(i == GRID - 1)
    def _finalize():
        loss_ref[...] = loss_ref[...] * (1.25 / float(N_ROWS * DIM))


_tc_call = pl.pallas_call(
    _tc_body,
    grid=(GRID,),
    in_specs=[
        pl.BlockSpec((BLOCK, DIM), lambda i: (i, 0)),
        pl.BlockSpec((N_EMB, DIM), lambda i: (0, 0)),
        pl.BlockSpec((DIM, N_EMB), lambda i: (0, 0)),
    ],
    out_specs=[
        pl.BlockSpec((BLOCK,), lambda i: (i,)),
        pl.BlockSpec((1, 1), lambda i: (0, 0)),
    ],
    out_shape=[
        jax.ShapeDtypeStruct((N_ROWS,), jnp.int32),
        jax.ShapeDtypeStruct((1, 1), jnp.float32),
    ],
)


def _sc_gather_body(table_hbm, idx_hbm, out_hbm, idx_v, rows_v, sem):
    wid = lax.axis_index("s") * SC_CORES + lax.axis_index("c")
    base = wid * ROWS_PER_WORKER

    def step(j, carry):
        off = pl.multiple_of(base + j * GATHER_CHUNK, GATHER_CHUNK)
        pltpu.sync_copy(idx_hbm.at[pl.ds(off, GATHER_CHUNK)], idx_v)
        pltpu.async_copy(table_hbm.at[idx_v], rows_v, sem).wait()
        pltpu.sync_copy(rows_v, out_hbm.at[pl.ds(off, GATHER_CHUNK)])
        return carry

    lax.fori_loop(0, CHUNKS_PER_WORKER, step, 0)


@functools.cache
def _sc_gather():
    return functools.partial(
        pl.kernel,
        out_type=jax.ShapeDtypeStruct((N_ROWS, DIM_PAD), jnp.float32),
        mesh=plsc.VectorSubcoreMesh(core_axis_name="c", subcore_axis_name="s"),
        scratch_types=[
            pltpu.VMEM((GATHER_CHUNK,), jnp.int32),
            pltpu.VMEM((GATHER_CHUNK, DIM_PAD), jnp.float32),
            pltpu.SemaphoreType.DMA,
        ],
    )(_sc_gather_body)


def kernel(inputs, embeddings):
    x = inputs.reshape(N_ROWS, DIM)
    et = embeddings.T
    idx, loss = _tc_call(x, embeddings, et)
    table = jnp.pad(embeddings, ((0, 0), (0, DIM_PAD - DIM)))
    latent = _sc_gather()(table, idx)[:, :DIM]
    return loss[0, 0], latent.reshape(inputs.shape)
